# baseline (speedup 1.0000x reference)
import jax
import jax.numpy as jnp
from jax import lax
from jax.experimental import pallas as pl
from jax.experimental.pallas import tpu as pltpu
from jax.experimental.pallas import tpu_sc as plsc

DIM = 64
GAMMA = 12.0
N_REL = 1000
N_ENT = 1000000
NC = 2
NS = 16
NW = NC * NS
L = 16

B = 16384
BPW = B // NW
CHUNK = 128
NCH = BPW // CHUNK
GROUPS = CHUNK // L
NJ = DIM // L


def _body(ent_hbm, rel_hbm, w_hbm, h_hbm, r_hbm, t_hbm, out_hbm,
          hi_v, ti_v, ri_v, rm_v, he_v, te_v, wv_v, rv_v,
          out_v, sem0, sem1):
    wid = lax.axis_index("s") * NC + lax.axis_index("c")
    base0 = wid * BPW
    lane = lax.iota(jnp.int32, L)
    sems = (sem0, sem1)

    def fire(slot, c):
        base = base0 + c * CHUNK
        sem = sems[slot]
        pltpu.sync_copy(h_hbm.at[pl.ds(base, CHUNK)], hi_v.at[slot])
        pltpu.sync_copy(t_hbm.at[pl.ds(base, CHUNK)], ti_v.at[slot])
        pltpu.sync_copy(r_hbm.at[pl.ds(base, CHUNK)], ri_v.at[slot])

        def idx_group(g, _):
            sl = pl.ds(g * L, L)
            r16 = ri_v[slot, sl]
            rm_v[slot, sl] = jnp.where(r16 >= N_REL, r16 - N_REL, r16)
            return 0

        lax.fori_loop(0, GROUPS, idx_group, 0)
        pltpu.async_copy(ent_hbm.at[hi_v.at[slot]], he_v.at[slot], sem)
        pltpu.async_copy(ent_hbm.at[ti_v.at[slot]], te_v.at[slot], sem)
        pltpu.async_copy(w_hbm.at[rm_v.at[slot]], wv_v.at[slot], sem)
        pltpu.async_copy(rel_hbm.at[rm_v.at[slot]], rv_v.at[slot], sem)

    def drain(slot):
        sem = sems[slot]
        pltpu.make_async_copy(ent_hbm.at[hi_v.at[slot]], he_v.at[slot], sem).wait()
        pltpu.make_async_copy(ent_hbm.at[ti_v.at[slot]], te_v.at[slot], sem).wait()
        pltpu.make_async_copy(w_hbm.at[rm_v.at[slot]], wv_v.at[slot], sem).wait()
        pltpu.make_async_copy(rel_hbm.at[rm_v.at[slot]], rv_v.at[slot], sem).wait()

    def compute(slot, c):
        def group(g, _):
            score_vec = jnp.zeros((L,), jnp.float32)
            sl = pl.ds(g * L, L)
            r16 = ri_v[slot, sl]
            sg16 = jnp.where(r16 >= N_REL, jnp.float32(-1.0),
                             jnp.float32(1.0))
            for k in range(L):
                i = g * L + k
                w_s = [wv_v[slot, i, pl.ds(j * L, L)] for j in range(NJ)]
                e_s = [he_v[slot, i, pl.ds(j * L, L)]
                       - te_v[slot, i, pl.ds(j * L, L)]
                       for j in range(NJ)]
                ww = w_s[0] * w_s[0]
                ew = e_s[0] * w_s[0]
                for j in range(1, NJ):
                    ww = ww + w_s[j] * w_s[j]
                    ew = ew + e_s[j] * w_s[j]
                s2_v = jnp.maximum(jnp.full((L,), jnp.sum(ww)),
                                   jnp.float32(1e-24))
                alpha = jnp.full((L,), jnp.sum(ew)) / s2_v
                sg = jnp.full((L,), sg16[k])
                acc = jnp.zeros((L,), jnp.float32)
                for j in range(NJ):
                    d = (e_s[j] - alpha * w_s[j]
                         + sg * rv_v[slot, i, pl.ds(j * L, L)])
                    acc = acc + jnp.abs(d)
                score = jnp.float32(GAMMA) - jnp.sum(acc)
                score_vec = jnp.where(lane == k, jnp.full((L,), score),
                                      score_vec)
            out_v[pl.ds(g * L, L)] = score_vec
            return 0

        lax.fori_loop(0, GROUPS, group, 0)
        pltpu.sync_copy(out_v, out_hbm.at[pl.ds(base0 + c * CHUNK, CHUNK)])

    fire(0, 0)
    for c in range(NCH):
        slot = c % 2
        if c + 1 < NCH:
            fire((c + 1) % 2, c + 1)
        drain(slot)
        compute(slot, c)


@jax.jit
def _transh_sc(ent2, rel_weight, w_weight, h, r, t):
    mesh = plsc.VectorSubcoreMesh(
        core_axis_name="c", subcore_axis_name="s", num_cores=NC, num_subcores=NS
    )
    kfn = pl.kernel(
        _body,
        out_type=jax.ShapeDtypeStruct((B,), jnp.float32),
        mesh=mesh,
        scratch_types=[
            pltpu.VMEM((2, CHUNK), jnp.int32),
            pltpu.VMEM((2, CHUNK), jnp.int32),
            pltpu.VMEM((2, CHUNK), jnp.int32),
            pltpu.VMEM((2, CHUNK), jnp.int32),
            pltpu.VMEM((2, CHUNK, 2 * DIM), jnp.float32),
            pltpu.VMEM((2, CHUNK, 2 * DIM), jnp.float32),
            pltpu.VMEM((2, CHUNK, DIM), jnp.float32),
            pltpu.VMEM((2, CHUNK, DIM), jnp.float32),
            pltpu.VMEM((CHUNK,), jnp.float32),
            pltpu.SemaphoreType.DMA,
            pltpu.SemaphoreType.DMA,
        ],
        compiler_params=pltpu.CompilerParams(
            needs_layout_passes=False, use_tc_tiling_on_sc=False
        ),
    )
    return kfn(ent2, rel_weight, w_weight, h, r, t)


def kernel(ent_weight, rel_weight, w_weight, h, r, t):
    ent2 = jnp.pad(ent_weight, ((0, 0), (0, DIM)))
    return _transh_sc(ent2, rel_weight, w_weight, h, r, t)


# speedup vs baseline: 1.3731x; 1.3731x over previous
import jax
import jax.numpy as jnp
from jax import lax
from jax.experimental import pallas as pl
from jax.experimental.pallas import tpu as pltpu
from jax.experimental.pallas import tpu_sc as plsc

DIM = 64
GAMMA = 12.0
N_REL = 1000
N_ENT = 1000000
NC = 2
NS = 16
NW = NC * NS
L = 16

B = 16384
BPW = B // NW
CHUNK = 16
NCH = BPW // CHUNK
GROUPS = CHUNK // L
NJ = DIM // L


def _body(ent_hbm, rel_hbm, w_hbm, h_hbm, r_hbm, t_hbm, out_hbm,
          hi_v, ti_v, ri_v, rm_v, he_v, te_v, wv_v, rv_v,
          out_v, sem0, sem1):
    wid = lax.axis_index("s") * NC + lax.axis_index("c")
    base0 = wid * BPW
    lane = lax.iota(jnp.int32, L)
    sems = (sem0, sem1)

    def fire(slot, c):
        base = pl.multiple_of(base0 + c * CHUNK, CHUNK)
        sem = sems[slot]
        pltpu.sync_copy(h_hbm.at[pl.ds(base, CHUNK)], hi_v.at[slot])
        pltpu.sync_copy(t_hbm.at[pl.ds(base, CHUNK)], ti_v.at[slot])
        pltpu.sync_copy(r_hbm.at[pl.ds(base, CHUNK)], ri_v.at[slot])

        for g in range(GROUPS):
            sl = pl.ds(g * L, L)
            r16 = ri_v[slot, sl]
            rm_v[slot, sl] = jnp.where(r16 >= N_REL, r16 - N_REL, r16)
            h16 = hi_v[slot, sl]
            t16 = ti_v[slot, sl]
            for k in range(L):
                i = g * L + k
                pltpu.async_copy(ent_hbm.at[pl.ds(h16[k], 1)],
                                 he_v.at[slot, i], sem)
                pltpu.async_copy(ent_hbm.at[pl.ds(t16[k], 1)],
                                 te_v.at[slot, i], sem)
        pltpu.async_copy(w_hbm.at[rm_v.at[slot]], wv_v.at[slot], sem)
        pltpu.async_copy(rel_hbm.at[rm_v.at[slot]], rv_v.at[slot], sem)

    def drain(slot):
        sem = sems[slot]
        for i in range(CHUNK):
            pltpu.make_async_copy(ent_hbm.at[pl.ds(0, 1)],
                                  he_v.at[slot, i], sem).wait()
            pltpu.make_async_copy(ent_hbm.at[pl.ds(0, 1)],
                                  te_v.at[slot, i], sem).wait()
        pltpu.make_async_copy(w_hbm.at[rm_v.at[slot]], wv_v.at[slot], sem).wait()
        pltpu.make_async_copy(rel_hbm.at[rm_v.at[slot]], rv_v.at[slot], sem).wait()

    def compute(slot, c):
        for g in range(GROUPS):
            score_vec = jnp.zeros((L,), jnp.float32)
            sl = pl.ds(g * L, L)
            r16 = ri_v[slot, sl]
            h16 = hi_v[slot, sl]
            t16 = ti_v[slot, sl]
            sg16 = jnp.where(r16 >= N_REL, jnp.float32(-1.0),
                             jnp.float32(1.0))
            for k in range(L):
                i = g * L + k
                w_s = [wv_v[slot, i, pl.ds(j * L, L)] for j in range(NJ)]
                e_s = [he_v[slot, i, 0, pl.ds(j * L, L)]
                       - te_v[slot, i, 0, pl.ds(j * L, L)]
                       for j in range(NJ)]
                ww = w_s[0] * w_s[0]
                ew = e_s[0] * w_s[0]
                for j in range(1, NJ):
                    ww = ww + w_s[j] * w_s[j]
                    ew = ew + e_s[j] * w_s[j]
                s2_v = jnp.maximum(jnp.full((L,), jnp.sum(ww)),
                                   jnp.float32(1e-24))
                alpha = jnp.full((L,), jnp.sum(ew)) / s2_v
                sg = jnp.full((L,), sg16[k])
                acc = jnp.zeros((L,), jnp.float32)
                for j in range(NJ):
                    d = (e_s[j] - alpha * w_s[j]
                         + sg * rv_v[slot, i, pl.ds(j * L, L)])
                    acc = acc + jnp.abs(d)
                score = jnp.float32(GAMMA) - jnp.sum(acc)
                score_vec = jnp.where(lane == k, jnp.full((L,), score),
                                      score_vec)
            out_v[pl.ds(g * L, L)] = score_vec
        ob = pl.multiple_of(base0 + c * CHUNK, CHUNK)
        pltpu.sync_copy(out_v, out_hbm.at[pl.ds(ob, CHUNK)])

    fire(0, 0)

    def pair_step(p, _):
        c0 = 2 * p
        c1 = c0 + 1
        fire(1, c1)
        drain(0)
        compute(0, c0)

        @pl.when(c1 + 1 < NCH)
        def _():
            fire(0, c1 + 1)

        drain(1)
        compute(1, c1)
        return 0

    lax.fori_loop(0, NCH // 2, pair_step, 0)


@jax.jit
def _transh_sc(ent_weight, rel2, w2, h, r, t):
    mesh = plsc.VectorSubcoreMesh(
        core_axis_name="c", subcore_axis_name="s", num_cores=NC, num_subcores=NS
    )
    kfn = pl.kernel(
        _body,
        out_type=jax.ShapeDtypeStruct((B,), jnp.float32),
        mesh=mesh,
        scratch_types=[
            pltpu.VMEM((2, CHUNK), jnp.int32),
            pltpu.VMEM((2, CHUNK), jnp.int32),
            pltpu.VMEM((2, CHUNK), jnp.int32),
            pltpu.VMEM((2, CHUNK), jnp.int32),
            pltpu.VMEM((2, CHUNK, 1, DIM), jnp.float32),
            pltpu.VMEM((2, CHUNK, 1, DIM), jnp.float32),
            pltpu.VMEM((2, CHUNK, 2 * DIM), jnp.float32),
            pltpu.VMEM((2, CHUNK, 2 * DIM), jnp.float32),
            pltpu.VMEM((CHUNK,), jnp.float32),
            pltpu.SemaphoreType.DMA,
            pltpu.SemaphoreType.DMA,
        ],
        compiler_params=pltpu.CompilerParams(
            needs_layout_passes=False, use_tc_tiling_on_sc=True
        ),
    )
    return kfn(ent_weight, rel2, w2, h, r, t)


def kernel(ent_weight, rel_weight, w_weight, h, r, t):
    rel2 = jnp.pad(rel_weight, ((0, 0), (0, DIM)))
    w2 = jnp.pad(w_weight, ((0, 0), (0, DIM)))
    return _transh_sc(ent_weight, rel2, w2, h, r, t)


# speedup vs baseline: 1.3747x; 1.0012x over previous
import jax
import jax.numpy as jnp
from jax import lax
from jax.experimental import pallas as pl
from jax.experimental.pallas import tpu as pltpu
from jax.experimental.pallas import tpu_sc as plsc

DIM = 64
GAMMA = 12.0
N_REL = 1000
N_ENT = 1000000
NC = 2
NS = 16
NW = NC * NS
L = 16

B = 16384
BPW = B // NW
CHUNK = 32
NCH = BPW // CHUNK
GROUPS = CHUNK // L
NJ = DIM // L


def _body(ent_hbm, rel_hbm, w_hbm, h_hbm, r_hbm, t_hbm, out_hbm,
          hi_v, ti_v, ri_v, rm_v, he_v, te_v, wv_v, rv_v,
          out_v, sem0, sem1):
    wid = lax.axis_index("s") * NC + lax.axis_index("c")
    base0 = wid * BPW
    lane = lax.iota(jnp.int32, L)
    sems = (sem0, sem1)

    def fire(slot, c):
        base = pl.multiple_of(base0 + c * CHUNK, CHUNK)
        sem = sems[slot]
        pltpu.sync_copy(h_hbm.at[pl.ds(base, CHUNK)], hi_v.at[slot])
        pltpu.sync_copy(t_hbm.at[pl.ds(base, CHUNK)], ti_v.at[slot])
        pltpu.sync_copy(r_hbm.at[pl.ds(base, CHUNK)], ri_v.at[slot])

        for g in range(GROUPS):
            sl = pl.ds(g * L, L)
            r16 = ri_v[slot, sl]
            rm_v[slot, sl] = jnp.where(r16 >= N_REL, r16 - N_REL, r16)
            h16 = hi_v[slot, sl]
            t16 = ti_v[slot, sl]
            for k in range(L):
                i = g * L + k
                pltpu.async_copy(ent_hbm.at[pl.ds(h16[k], 1)],
                                 he_v.at[slot, i], sem)
                pltpu.async_copy(ent_hbm.at[pl.ds(t16[k], 1)],
                                 te_v.at[slot, i], sem)
        pltpu.async_copy(w_hbm.at[rm_v.at[slot]], wv_v.at[slot], sem)
        pltpu.async_copy(rel_hbm.at[rm_v.at[slot]], rv_v.at[slot], sem)

    def drain(slot):
        sem = sems[slot]
        for i in range(CHUNK):
            pltpu.make_async_copy(ent_hbm.at[pl.ds(0, 1)],
                                  he_v.at[slot, i], sem).wait()
            pltpu.make_async_copy(ent_hbm.at[pl.ds(0, 1)],
                                  te_v.at[slot, i], sem).wait()
        pltpu.make_async_copy(w_hbm.at[rm_v.at[slot]], wv_v.at[slot], sem).wait()
        pltpu.make_async_copy(rel_hbm.at[rm_v.at[slot]], rv_v.at[slot], sem).wait()

    def compute(slot, c):
        for g in range(GROUPS):
            score_vec = jnp.zeros((L,), jnp.float32)
            sl = pl.ds(g * L, L)
            r16 = ri_v[slot, sl]
            h16 = hi_v[slot, sl]
            t16 = ti_v[slot, sl]
            sg16 = jnp.where(r16 >= N_REL, jnp.float32(-1.0),
                             jnp.float32(1.0))
            for k in range(L):
                i = g * L + k
                w_s = [wv_v[slot, i, pl.ds(j * L, L)] for j in range(NJ)]
                e_s = [he_v[slot, i, 0, pl.ds(j * L, L)]
                       - te_v[slot, i, 0, pl.ds(j * L, L)]
                       for j in range(NJ)]
                ww = w_s[0] * w_s[0]
                ew = e_s[0] * w_s[0]
                for j in range(1, NJ):
                    ww = ww + w_s[j] * w_s[j]
                    ew = ew + e_s[j] * w_s[j]
                s2_v = jnp.maximum(jnp.full((L,), jnp.sum(ww)),
                                   jnp.float32(1e-24))
                alpha = jnp.full((L,), jnp.sum(ew)) / s2_v
                sg = jnp.full((L,), sg16[k])
                acc = jnp.zeros((L,), jnp.float32)
                for j in range(NJ):
                    d = (e_s[j] - alpha * w_s[j]
                         + sg * rv_v[slot, i, pl.ds(j * L, L)])
                    acc = acc + jnp.abs(d)
                score = jnp.float32(GAMMA) - jnp.sum(acc)
                score_vec = jnp.where(lane == k, jnp.full((L,), score),
                                      score_vec)
            out_v[pl.ds(g * L, L)] = score_vec
        ob = pl.multiple_of(base0 + c * CHUNK, CHUNK)
        pltpu.sync_copy(out_v, out_hbm.at[pl.ds(ob, CHUNK)])

    fire(0, 0)

    def pair_step(p, _):
        c0 = 2 * p
        c1 = c0 + 1
        fire(1, c1)
        drain(0)
        compute(0, c0)

        @pl.when(c1 + 1 < NCH)
        def _():
            fire(0, c1 + 1)

        drain(1)
        compute(1, c1)
        return 0

    lax.fori_loop(0, NCH // 2, pair_step, 0)


@jax.jit
def _transh_sc(ent_weight, rel2, w2, h, r, t):
    mesh = plsc.VectorSubcoreMesh(
        core_axis_name="c", subcore_axis_name="s", num_cores=NC, num_subcores=NS
    )
    kfn = pl.kernel(
        _body,
        out_type=jax.ShapeDtypeStruct((B,), jnp.float32),
        mesh=mesh,
        scratch_types=[
            pltpu.VMEM((2, CHUNK), jnp.int32),
            pltpu.VMEM((2, CHUNK), jnp.int32),
            pltpu.VMEM((2, CHUNK), jnp.int32),
            pltpu.VMEM((2, CHUNK), jnp.int32),
            pltpu.VMEM((2, CHUNK, 1, DIM), jnp.float32),
            pltpu.VMEM((2, CHUNK, 1, DIM), jnp.float32),
            pltpu.VMEM((2, CHUNK, 2 * DIM), jnp.float32),
            pltpu.VMEM((2, CHUNK, 2 * DIM), jnp.float32),
            pltpu.VMEM((CHUNK,), jnp.float32),
            pltpu.SemaphoreType.DMA,
            pltpu.SemaphoreType.DMA,
        ],
        compiler_params=pltpu.CompilerParams(
            needs_layout_passes=False, use_tc_tiling_on_sc=True
        ),
    )
    return kfn(ent_weight, rel2, w2, h, r, t)


def kernel(ent_weight, rel_weight, w_weight, h, r, t):
    rel2 = jnp.pad(rel_weight, ((0, 0), (0, DIM)))
    w2 = jnp.pad(w_weight, ((0, 0), (0, DIM)))
    return _transh_sc(ent_weight, rel2, w2, h, r, t)
